# Initial kernel scaffold; baseline (speedup 1.0000x reference)
import jax
import jax.numpy as jnp
from jax.experimental import pallas as pl


def kernel(data, org_edge_index, emb_table, e_base, low_rank_u, low_rank_v, W_cond, b_cond, W_ap, b_ap, w_av, W_r, b_r, W_lin, att_i, att_j, att_em_i, att_em_j, gnn_bias, bn1_g, bn1_b, bn2_g, bn2_b, W_out, b_out):
    raise NotImplementedError("write your pallas kernel here")


# speedup vs baseline: 87.3644x; 87.3644x over previous
import functools

import jax
import jax.numpy as jnp
from jax import lax
from jax.experimental import pallas as pl

NODE_NUM = 512
DIM = 128
INPUT_DIM = 64
MOE = 8
RTK = 2
TOPK = 20
TAU = 1.0
B = 64
BN_ = B * NODE_NUM
NEG = -1e30


def _mm(a, b):
    return lax.dot_general(a, b, (((1,), (0,)), ((), ())),
                           preferred_element_type=jnp.float32)


def _mm_t(a, b):
    return lax.dot_general(a, b, (((1,), (1,)), ((), ())),
                           preferred_element_type=jnp.float32)


def _stage_a(data_ref, wc_ref, bc_ref, wap_ref, bap_ref, wav_ref, hsys_ref):
    x = data_ref[0]
    h = _mm(x, wc_ref[...]) + bc_ref[...]
    ah = _mm(h, wap_ref[...]) + bap_ref[...]
    ah = jnp.where(ah >= 0, ah, 0.2 * ah)
    e = _mm(ah, wav_ref[...])
    m = jnp.max(e, axis=0, keepdims=True)
    ex = jnp.exp(e - m)
    beta = ex / jnp.sum(ex, axis=0, keepdims=True)
    hsys_ref[0] = jnp.sum(beta * h, axis=0, keepdims=True)


def _stage_b(hsys_ref, wr_ref, br_ref, g_ref, psoft_ref, pt_ref):
    z = _mm(hsys_ref[...], wr_ref[...]) + br_ref[...]
    zg = (z + g_ref[...]) / TAU
    m = jnp.max(zg, axis=1, keepdims=True)
    ex = jnp.exp(zg - m)
    ps = ex / jnp.sum(ex, axis=1, keepdims=True)
    psoft_ref[...] = ps
    m1 = jnp.max(ps, axis=1, keepdims=True)
    nmax = jnp.sum(jnp.where(ps == m1, 1.0, 0.0), axis=1, keepdims=True)
    m2 = jnp.max(jnp.where(ps == m1, NEG, ps), axis=1, keepdims=True)
    thr = jnp.where(nmax > 1.5, m1, m2)
    ph = jnp.where(ps >= thr, ps, 0.0)
    pt_ref[...] = ph / jnp.maximum(jnp.sum(ph, axis=1, keepdims=True), 1e-12)


def _stage_c(u_ref, v_ref, eb_ref, pd_ref):
    pd_ref[0] = _mm(u_ref[0], v_ref[0]) + eb_ref[...]


def _stage_c2(pt_ref, pd_ref, mixed_ref):
    mixed_ref[...] = _mm(pt_ref[...], pd_ref[...])


def _stage_d(data_ref, mixed_ref, wl_ref, ai_ref, aj_ref, aei_ref, aej_ref,
             gb_ref, agg_ref, s1_ref, ss1_ref):
    b = pl.program_id(0)
    x = data_ref[0]
    mx = mixed_ref[0]
    xl = _mm(x, wl_ref[...])
    a_i = _mm(xl, ai_ref[...]) + _mm(mx, aei_ref[...])
    a_j = _mm_t(aj_ref[...], xl) + _mm_t(aej_ref[...], mx)
    scores = _mm_t(mx, mx)
    cur = scores
    rowmax = None
    t = None
    for i in range(TOPK):
        t = jnp.max(cur, axis=1, keepdims=True)
        if i == 0:
            rowmax = t
        if i < TOPK - 1:
            cur = jnp.where(cur >= t, NEG, cur)
    mask = scores >= t
    ews = jnp.where(mask, jnp.exp(scores - rowmax), 0.0)
    ew = ews / jnp.sum(ews, axis=1, keepdims=True)
    alpha = a_i + a_j
    alpha = jnp.where(alpha >= 0, alpha, 0.2 * alpha)
    amax = jnp.max(jnp.where(mask, alpha, NEG), axis=1, keepdims=True)
    exa = jnp.where(mask, jnp.exp(alpha - amax), 0.0)
    den = jnp.sum(exa, axis=1, keepdims=True)
    wmat = (exa / (den + 1e-16)) * ew
    agg = _mm(wmat, xl) + gb_ref[...]
    agg_ref[0] = agg

    @pl.when(b == 0)
    def _():
        s1_ref[...] = jnp.zeros_like(s1_ref)
        ss1_ref[...] = jnp.zeros_like(ss1_ref)

    s1_ref[...] += jnp.sum(agg, axis=0, keepdims=True)
    ss1_ref[...] += jnp.sum(agg * agg, axis=0, keepdims=True)


def _stage_e(agg_ref, s1_ref, ss1_ref, g1_ref, b1_ref, emb_ref,
             pre_ref, s2_ref, ss2_ref):
    b = pl.program_id(0)
    a = agg_ref[0]
    mean = s1_ref[...] * (1.0 / BN_)
    var = ss1_ref[...] * (1.0 / BN_) - mean * mean
    inv = lax.rsqrt(var + 1e-5)
    y = (a - mean) * inv * g1_ref[...] + b1_ref[...]
    y = jnp.maximum(y, 0.0)
    pre = y * emb_ref[...]
    pre_ref[0] = pre

    @pl.when(b == 0)
    def _():
        s2_ref[...] = jnp.zeros_like(s2_ref)
        ss2_ref[...] = jnp.zeros_like(ss2_ref)

    s2_ref[...] += jnp.sum(pre, axis=0, keepdims=True)
    ss2_ref[...] += jnp.sum(pre * pre, axis=0, keepdims=True)


def _stage_f(pre_ref, s2_ref, ss2_ref, g2_ref, b2_ref, wo_ref, bo_ref,
             out_ref):
    p = pre_ref[0]
    mean = s2_ref[...] * (1.0 / BN_)
    var = ss2_ref[...] * (1.0 / BN_) - mean * mean
    inv = lax.rsqrt(var + 1e-5)
    y = (p - mean) * inv * g2_ref[...] + b2_ref[...]
    y = jnp.maximum(y, 0.0)
    out_ref[0] = _mm_t(wo_ref[...], y) + bo_ref[...]


def kernel(data, org_edge_index, emb_table, e_base, low_rank_u, low_rank_v,
           W_cond, b_cond, W_ap, b_ap, w_av, W_r, b_r, W_lin, att_i, att_j,
           att_em_i, att_em_j, gnn_bias, bn1_g, bn1_b, bn2_g, bn2_b, W_out,
           b_out):
    f32 = jnp.float32
    N, D, F, M = NODE_NUM, DIM, INPUT_DIM, MOE
    row = lambda v: v.reshape(1, -1).astype(f32)
    col = lambda v: v.reshape(-1, 1).astype(f32)

    u = jnp.clip(jax.random.uniform(jax.random.key(42), (B, M), f32),
                 1e-6, 1.0 - 1e-6)
    g_const = -jnp.log(-jnp.log(u))

    h_sys = pl.pallas_call(
        _stage_a,
        grid=(B,),
        in_specs=[
            pl.BlockSpec((1, N, F), lambda b: (b, 0, 0)),
            pl.BlockSpec((F, D), lambda b: (0, 0)),
            pl.BlockSpec((1, D), lambda b: (0, 0)),
            pl.BlockSpec((D, D), lambda b: (0, 0)),
            pl.BlockSpec((1, D), lambda b: (0, 0)),
            pl.BlockSpec((D, 1), lambda b: (0, 0)),
        ],
        out_specs=pl.BlockSpec((1, 1, D), lambda b: (b, 0, 0)),
        out_shape=jax.ShapeDtypeStruct((B, 1, D), f32),
    )(data, W_cond, row(b_cond), W_ap, row(b_ap), col(w_av))
    h_sys = h_sys.reshape(B, D)

    pi_soft, pi_t = pl.pallas_call(
        _stage_b,
        in_specs=[pl.BlockSpec((B, D), lambda: (0, 0)),
                  pl.BlockSpec((D, M), lambda: (0, 0)),
                  pl.BlockSpec((1, M), lambda: (0, 0)),
                  pl.BlockSpec((B, M), lambda: (0, 0))],
        out_specs=[pl.BlockSpec((B, M), lambda: (0, 0)),
                   pl.BlockSpec((B, M), lambda: (0, 0))],
        out_shape=[jax.ShapeDtypeStruct((B, M), f32),
                   jax.ShapeDtypeStruct((B, M), f32)],
    )(h_sys, W_r, row(b_r), g_const)

    pd = pl.pallas_call(
        _stage_c,
        grid=(M,),
        in_specs=[pl.BlockSpec((1, N, 8), lambda m: (m, 0, 0)),
                  pl.BlockSpec((1, 8, D), lambda m: (m, 0, 0)),
                  pl.BlockSpec((N, D), lambda m: (0, 0))],
        out_specs=pl.BlockSpec((1, N, D), lambda m: (m, 0, 0)),
        out_shape=jax.ShapeDtypeStruct((M, N, D), f32),
    )(low_rank_u, low_rank_v, e_base)

    CH = 4096
    NC = N * D // CH
    mixed_flat = pl.pallas_call(
        _stage_c2,
        grid=(NC,),
        in_specs=[pl.BlockSpec((B, M), lambda c: (0, 0)),
                  pl.BlockSpec((M, CH), lambda c: (0, c))],
        out_specs=pl.BlockSpec((B, CH), lambda c: (0, c)),
        out_shape=jax.ShapeDtypeStruct((B, N * D), f32),
    )(pi_t, pd.reshape(M, N * D))
    mixed = mixed_flat.reshape(B, N, D)

    agg, s1, ss1 = pl.pallas_call(
        _stage_d,
        grid=(B,),
        in_specs=[
            pl.BlockSpec((1, N, F), lambda b: (b, 0, 0)),
            pl.BlockSpec((1, N, D), lambda b: (b, 0, 0)),
            pl.BlockSpec((F, D), lambda b: (0, 0)),
            pl.BlockSpec((D, 1), lambda b: (0, 0)),
            pl.BlockSpec((1, D), lambda b: (0, 0)),
            pl.BlockSpec((D, 1), lambda b: (0, 0)),
            pl.BlockSpec((1, D), lambda b: (0, 0)),
            pl.BlockSpec((1, D), lambda b: (0, 0)),
        ],
        out_specs=[pl.BlockSpec((1, N, D), lambda b: (b, 0, 0)),
                   pl.BlockSpec((1, D), lambda b: (0, 0)),
                   pl.BlockSpec((1, D), lambda b: (0, 0))],
        out_shape=[jax.ShapeDtypeStruct((B, N, D), f32),
                   jax.ShapeDtypeStruct((1, D), f32),
                   jax.ShapeDtypeStruct((1, D), f32)],
    )(data, mixed, W_lin, col(att_i), row(att_j), col(att_em_i),
      row(att_em_j), row(gnn_bias))

    pre, s2, ss2 = pl.pallas_call(
        _stage_e,
        grid=(B,),
        in_specs=[
            pl.BlockSpec((1, N, D), lambda b: (b, 0, 0)),
            pl.BlockSpec((1, D), lambda b: (0, 0)),
            pl.BlockSpec((1, D), lambda b: (0, 0)),
            pl.BlockSpec((1, D), lambda b: (0, 0)),
            pl.BlockSpec((1, D), lambda b: (0, 0)),
            pl.BlockSpec((N, D), lambda b: (0, 0)),
        ],
        out_specs=[pl.BlockSpec((1, N, D), lambda b: (b, 0, 0)),
                   pl.BlockSpec((1, D), lambda b: (0, 0)),
                   pl.BlockSpec((1, D), lambda b: (0, 0))],
        out_shape=[jax.ShapeDtypeStruct((B, N, D), f32),
                   jax.ShapeDtypeStruct((1, D), f32),
                   jax.ShapeDtypeStruct((1, D), f32)],
    )(agg, s1, ss1, row(bn1_g), row(bn1_b), emb_table)

    out = pl.pallas_call(
        _stage_f,
        grid=(B,),
        in_specs=[
            pl.BlockSpec((1, N, D), lambda b: (b, 0, 0)),
            pl.BlockSpec((1, D), lambda b: (0, 0)),
            pl.BlockSpec((1, D), lambda b: (0, 0)),
            pl.BlockSpec((1, D), lambda b: (0, 0)),
            pl.BlockSpec((1, D), lambda b: (0, 0)),
            pl.BlockSpec((1, D), lambda b: (0, 0)),
            pl.BlockSpec((1, 1), lambda b: (0, 0)),
        ],
        out_specs=pl.BlockSpec((1, 1, N), lambda b: (b, 0, 0)),
        out_shape=jax.ShapeDtypeStruct((B, 1, N), f32),
    )(pre, s2, ss2, row(bn2_g), row(bn2_b), row(W_out), b_out.reshape(1, 1))

    return out.reshape(B, N), h_sys, pi_soft
